# baseline (speedup 1.0000x reference)
import functools

import jax
import jax.numpy as jnp
from jax import lax
from jax.experimental import pallas as pl
from jax.experimental.pallas import tpu as pltpu
from jax.experimental.pallas import tpu_sc as plsc

_NC, _NS = 2, 16
_NW = _NC * _NS


def _sc_gather(idxs, tables, B, D):
    n_feat = len(tables)
    mesh = plsc.ScalarSubcoreMesh(axis_name="c", num_cores=_NC)

    @functools.partial(
        pl.kernel,
        out_type=jax.ShapeDtypeStruct((n_feat * B, D), jnp.float32),
        mesh=mesh,
        scratch_types=[
            pltpu.SMEM((B,), jnp.int32),
            pltpu.SemaphoreType.DMA,
        ],
    )
    def gather_kernel(*refs):
        idx_refs = refs[:n_feat]
        tbl_refs = refs[n_feat:2 * n_feat]
        out_hbm = refs[2 * n_feat]
        idx_s, sem = refs[2 * n_feat + 1:]
        core = lax.axis_index("c")
        for f in range(n_feat):
            tbl = tbl_refs[f]

            @pl.when(core == f % _NC)
            def _(f=f, tbl=tbl):
                pltpu.sync_copy(idx_refs[f], idx_s)

                @pl.loop(0, B)
                def _(i):
                    pltpu.async_copy(
                        tbl.at[pl.ds(idx_s[i], 1)],
                        out_hbm.at[pl.ds(f * B + i, 1)], sem)

                @pl.loop(0, B)
                def _(i):
                    pltpu.make_async_copy(
                        tbl.at[pl.ds(0, 1)],
                        out_hbm.at[pl.ds(f * B, 1)], sem).wait()

    return gather_kernel(*idxs, *tables)


def _mlp_body(x_ref, w1_ref, b1_ref, w2_ref, b2_ref, w3_ref, b3_ref,
              pid_ref, uid_ref, ww_ref, wb_ref, d_ref, w_ref):
    hi = jax.lax.Precision.HIGHEST
    x = x_ref[...]
    h = jnp.dot(x, w1_ref[...], precision=hi, preferred_element_type=jnp.float32)
    h = jnp.maximum(h + b1_ref[...], 0.0)
    h = jnp.dot(h, w2_ref[...], precision=hi, preferred_element_type=jnp.float32)
    h = jnp.maximum(h + b2_ref[...], 0.0)
    d = jnp.dot(h, w3_ref[...], precision=hi, preferred_element_type=jnp.float32)
    d_ref[...] = d + b3_ref[...]
    ww = ww_ref[...]
    w_ref[...] = (pid_ref[...] * ww[0:1, 0:1] + uid_ref[...] * ww[0:1, 1:2]
                  + wb_ref[...])


def _writer_body(d_ref, w_ref, o_ref):
    dv = d_ref[...]
    wv = w_ref[...]
    t = dv[:, None, :] + wv[None, :, :]
    o_ref[...] = 0.5 * jnp.tanh(0.5 * t) + 0.5


def kernel(product_id, user_id, year, month, day_of_week, hour,
           product_table, user_table, year_table, month_table, dow_table,
           hour_table, wide_W, wide_b, W1, b1, W2, b2, W3, b3):
    B = product_id.shape[0]
    D = product_table.shape[1]
    n_feat = 6

    idxs = [a.reshape(-1).astype(jnp.int32)
            for a in (product_id, user_id, year, month, day_of_week, hour)]
    tables = (product_table, user_table, year_table, month_table, dow_table,
              hour_table)

    deep_in = _sc_gather(idxs, tables, B, D)

    d, w = pl.pallas_call(
        _mlp_body,
        out_shape=(
            jax.ShapeDtypeStruct((n_feat * B, W3.shape[0]), jnp.float32),
            jax.ShapeDtypeStruct((B, 1), jnp.float32),
        ),
    )(deep_in, W1.T, b1.reshape(1, -1), W2.T, b2.reshape(1, -1), W3.T,
      b3.reshape(1, -1), product_id.astype(jnp.float32),
      user_id.astype(jnp.float32), wide_W, wide_b.reshape(1, 1))

    BI = 16
    n_out = W3.shape[0]
    out = pl.pallas_call(
        _writer_body,
        grid=(n_feat * B // BI,),
        in_specs=[
            pl.BlockSpec((BI, n_out), lambda i: (i, 0)),
            pl.BlockSpec((B, 1), lambda i: (0, 0)),
        ],
        out_specs=pl.BlockSpec((BI, B, n_out), lambda i: (i, 0, 0)),
        out_shape=jax.ShapeDtypeStruct((n_feat * B, B, n_out), jnp.float32),
    )(d, w)
    return out


# speedup vs baseline: 1.0004x; 1.0004x over previous
import functools

import jax
import jax.numpy as jnp
from jax import lax
from jax.experimental import pallas as pl
from jax.experimental.pallas import tpu as pltpu
from jax.experimental.pallas import tpu_sc as plsc

_NC, _NS = 2, 16
_NW = _NC * _NS


def _sc_gather(idxs, tables, B, D):
    n_feat = len(tables)
    mesh = plsc.ScalarSubcoreMesh(axis_name="c", num_cores=_NC)

    @functools.partial(
        pl.kernel,
        out_type=jax.ShapeDtypeStruct((n_feat * B, D), jnp.float32),
        mesh=mesh,
        scratch_types=[
            pltpu.SMEM((B,), jnp.int32),
            pltpu.SemaphoreType.DMA,
        ],
    )
    def gather_kernel(*refs):
        idx_refs = refs[:n_feat]
        tbl_refs = refs[n_feat:2 * n_feat]
        out_hbm = refs[2 * n_feat]
        idx_s, sem = refs[2 * n_feat + 1:]
        core = lax.axis_index("c")
        for f in range(n_feat):
            tbl = tbl_refs[f]

            @pl.when(core == f % _NC)
            def _(f=f, tbl=tbl):
                pltpu.sync_copy(idx_refs[f], idx_s)

                @pl.loop(0, B)
                def _(i):
                    pltpu.async_copy(
                        tbl.at[pl.ds(idx_s[i], 1)],
                        out_hbm.at[pl.ds(f * B + i, 1)], sem)

                @pl.loop(0, B)
                def _(i):
                    pltpu.make_async_copy(
                        tbl.at[pl.ds(0, 1)],
                        out_hbm.at[pl.ds(f * B, 1)], sem).wait()

    return gather_kernel(*idxs, *tables)


def _mlp_body(x_ref, w1_ref, b1_ref, w2_ref, b2_ref, w3_ref, b3_ref,
              pid_ref, uid_ref, ww_ref, wb_ref, d_ref, w_ref):
    x = x_ref[...].astype(jnp.bfloat16)
    h = jnp.dot(x, w1_ref[...], preferred_element_type=jnp.float32)
    h = jnp.maximum(h + b1_ref[...], 0.0).astype(jnp.bfloat16)
    h = jnp.dot(h, w2_ref[...], preferred_element_type=jnp.float32)
    h = jnp.maximum(h + b2_ref[...], 0.0).astype(jnp.bfloat16)
    d = jnp.dot(h, w3_ref[...], preferred_element_type=jnp.float32)
    d_ref[...] = d + b3_ref[...]
    ww = ww_ref[...]
    w_ref[...] = (pid_ref[...] * ww[0:1, 0:1] + uid_ref[...] * ww[0:1, 1:2]
                  + wb_ref[...])


def _writer_body(d_ref, w_ref, o_ref):
    dv = d_ref[...]
    wv = w_ref[...]
    t = dv[:, None, :] + wv[None, :, :]
    o_ref[...] = 0.5 * jnp.tanh(0.5 * t) + 0.5


def kernel(product_id, user_id, year, month, day_of_week, hour,
           product_table, user_table, year_table, month_table, dow_table,
           hour_table, wide_W, wide_b, W1, b1, W2, b2, W3, b3):
    B = product_id.shape[0]
    D = product_table.shape[1]
    n_feat = 6

    idxs = [a.reshape(-1).astype(jnp.int32)
            for a in (product_id, user_id, year, month, day_of_week, hour)]
    tables = (product_table, user_table, year_table, month_table, dow_table,
              hour_table)

    deep_in = _sc_gather(idxs, tables, B, D)

    d, w = pl.pallas_call(
        _mlp_body,
        out_shape=(
            jax.ShapeDtypeStruct((n_feat * B, W3.shape[0]), jnp.float32),
            jax.ShapeDtypeStruct((B, 1), jnp.float32),
        ),
    )(deep_in, W1.T.astype(jnp.bfloat16), b1.reshape(1, -1),
      W2.T.astype(jnp.bfloat16), b2.reshape(1, -1),
      W3.T.astype(jnp.bfloat16), b3.reshape(1, -1),
      product_id.astype(jnp.float32),
      user_id.astype(jnp.float32), wide_W, wide_b.reshape(1, 1))

    BI = 16
    n_out = W3.shape[0]
    out = pl.pallas_call(
        _writer_body,
        grid=(n_feat * B // BI,),
        in_specs=[
            pl.BlockSpec((BI, n_out), lambda i: (i, 0)),
            pl.BlockSpec((B, 1), lambda i: (0, 0)),
        ],
        out_specs=pl.BlockSpec((BI, B, n_out), lambda i: (i, 0, 0)),
        out_shape=jax.ShapeDtypeStruct((n_feat * B, B, n_out), jnp.float32),
    )(d, w)
    return out


# speedup vs baseline: 1.0011x; 1.0007x over previous
import functools

import jax
import jax.numpy as jnp
from jax import lax
from jax.experimental import pallas as pl
from jax.experimental.pallas import tpu as pltpu
from jax.experimental.pallas import tpu_sc as plsc

_NC, _NS = 2, 16
_NW = _NC * _NS


def _sc_gather(idxs, tables, B, D):
    n_feat = len(tables)
    mesh = plsc.ScalarSubcoreMesh(axis_name="c", num_cores=_NC)

    @functools.partial(
        pl.kernel,
        out_type=jax.ShapeDtypeStruct((n_feat * B, D), jnp.float32),
        mesh=mesh,
        scratch_types=[
            pltpu.SMEM((B,), jnp.int32),
            pltpu.SemaphoreType.DMA,
        ],
    )
    def gather_kernel(*refs):
        idx_refs = refs[:n_feat]
        tbl_refs = refs[n_feat:2 * n_feat]
        out_hbm = refs[2 * n_feat]
        idx_s, sem = refs[2 * n_feat + 1:]
        core = lax.axis_index("c")
        for f in range(n_feat):
            tbl = tbl_refs[f]

            @pl.when(core == f % _NC)
            def _(f=f, tbl=tbl):
                pltpu.sync_copy(idx_refs[f], idx_s)

                @pl.loop(0, B)
                def _(i):
                    pltpu.async_copy(
                        tbl.at[pl.ds(idx_s[i], 1)],
                        out_hbm.at[pl.ds(f * B + i, 1)], sem)

                pltpu.make_async_copy(
                    tbl.at[pl.ds(0, B)] if tbl.shape[0] >= B
                    else out_hbm.at[pl.ds(0, B)],
                    out_hbm.at[pl.ds(f * B, B)], sem).wait()

    return gather_kernel(*idxs, *tables)


def _mlp_body(x_ref, w1_ref, b1_ref, w2_ref, b2_ref, w3_ref, b3_ref,
              pid_ref, uid_ref, ww_ref, wb_ref, d_ref, w_ref):
    x = x_ref[...].astype(jnp.bfloat16)
    h = jnp.dot(x, w1_ref[...], preferred_element_type=jnp.float32)
    h = jnp.maximum(h + b1_ref[...], 0.0).astype(jnp.bfloat16)
    h = jnp.dot(h, w2_ref[...], preferred_element_type=jnp.float32)
    h = jnp.maximum(h + b2_ref[...], 0.0).astype(jnp.bfloat16)
    d = jnp.dot(h, w3_ref[...], preferred_element_type=jnp.float32)
    d_ref[...] = d + b3_ref[...]
    ww = ww_ref[...].astype(jnp.bfloat16).astype(jnp.float32)
    pb = pid_ref[...].astype(jnp.bfloat16).astype(jnp.float32)
    ub = uid_ref[...].astype(jnp.bfloat16).astype(jnp.float32)
    w_ref[...] = pb * ww[0:1, 0:1] + ub * ww[0:1, 1:2] + wb_ref[...]


def _writer_body(d_ref, w_ref, o_ref):
    dv = d_ref[...]
    wv = w_ref[...]
    t = dv[:, None, :] + wv[None, :, :]
    o_ref[...] = 0.5 * jnp.tanh(0.5 * t) + 0.5


def kernel(product_id, user_id, year, month, day_of_week, hour,
           product_table, user_table, year_table, month_table, dow_table,
           hour_table, wide_W, wide_b, W1, b1, W2, b2, W3, b3):
    B = product_id.shape[0]
    D = product_table.shape[1]
    n_feat = 6

    idxs = [a.reshape(-1).astype(jnp.int32)
            for a in (product_id, user_id, year, month, day_of_week, hour)]
    tables = (product_table, user_table, year_table, month_table, dow_table,
              hour_table)

    deep_in = _sc_gather(idxs, tables, B, D)

    d, w = pl.pallas_call(
        _mlp_body,
        out_shape=(
            jax.ShapeDtypeStruct((n_feat * B, W3.shape[0]), jnp.float32),
            jax.ShapeDtypeStruct((B, 1), jnp.float32),
        ),
    )(deep_in, W1.T.astype(jnp.bfloat16), b1.reshape(1, -1),
      W2.T.astype(jnp.bfloat16), b2.reshape(1, -1),
      W3.T.astype(jnp.bfloat16), b3.reshape(1, -1),
      product_id.astype(jnp.float32),
      user_id.astype(jnp.float32), wide_W, wide_b.reshape(1, 1))

    BI = 16
    n_out = W3.shape[0]
    out = pl.pallas_call(
        _writer_body,
        grid=(n_feat * B // BI,),
        in_specs=[
            pl.BlockSpec((BI, n_out), lambda i: (i, 0)),
            pl.BlockSpec((B, 1), lambda i: (0, 0)),
        ],
        out_specs=pl.BlockSpec((BI, B, n_out), lambda i: (i, 0, 0)),
        out_shape=jax.ShapeDtypeStruct((n_feat * B, B, n_out), jnp.float32),
    )(d, w)
    return out
